# Initial kernel scaffold; baseline (speedup 1.0000x reference)
import jax
import jax.numpy as jnp
from jax.experimental import pallas as pl


def kernel(seq1M, seq2M, patches, geo):
    raise NotImplementedError("write your pallas kernel here")


# speedup vs baseline: 1.7273x; 1.7273x over previous
import jax
import jax.numpy as jnp
from jax.experimental import pallas as pl
from jax.experimental.pallas import tpu as pltpu

B = 32
P = 16
PS = 32
D = 64
SR = 2048
SL = 1024
CH = 2 * D + 1


def _tc_body(patches_s, seq1_ref, seq2_ref, geo_ref, out_ref):
    i = pl.program_id(0)
    p = pl.program_id(1)
    r = patches_s[i, p, 0]
    c = patches_s[i, p, 1]
    row = seq1_ref[0, pl.ds(r, PS), :]
    col = seq2_ref[0, pl.ds(c, PS), :]
    row_t = jnp.broadcast_to(row[None, :, :], (PS, PS, D))
    col_t = jnp.broadcast_to(col[:, None, :], (PS, PS, D))
    g = geo_ref[0, 0][:, :, None]
    out_ref[0, 0] = jnp.concatenate([row_t, col_t, g], axis=-1)


def kernel(seq1M, seq2M, patches, geo):
    grid_spec = pltpu.PrefetchScalarGridSpec(
        num_scalar_prefetch=1,
        grid=(B, P),
        in_specs=[
            pl.BlockSpec((1, SR, D), lambda i, p, s: (i, 0, 0)),
            pl.BlockSpec((1, SL, D), lambda i, p, s: (i, 0, 0)),
            pl.BlockSpec((1, 1, PS, PS), lambda i, p, s: (i, p, 0, 0)),
        ],
        out_specs=pl.BlockSpec(
            (1, 1, PS, PS, CH), lambda i, p, s: (i, p, 0, 0, 0)
        ),
    )
    return pl.pallas_call(
        _tc_body,
        grid_spec=grid_spec,
        out_shape=jax.ShapeDtypeStruct((B, P, PS, PS, CH), jnp.float32),
    )(patches, seq1M, seq2M, geo)


# speedup vs baseline: 2.8018x; 1.6220x over previous
import jax
import jax.numpy as jnp
from jax.experimental import pallas as pl
from jax.experimental.pallas import tpu as pltpu

B = 32
P = 16
PS = 32
D = 64
SR = 2048
SL = 1024
CH = 2 * D + 1


PP = 4


def _tc_body(patches_s, seq1_ref, seq2_ref, geo_ref, out_ref):
    i = pl.program_id(0)
    pb = pl.program_id(1)
    for k in range(PP):
        p = pb * PP + k
        r = patches_s[i, p, 0]
        c = patches_s[i, p, 1]
        row = seq1_ref[0, pl.ds(r, PS), :]
        col = seq2_ref[0, pl.ds(c, PS), :]
        row_t = jnp.broadcast_to(row[None, :, :], (PS, PS, D))
        col_t = jnp.broadcast_to(col[:, None, :], (PS, PS, D))
        g = geo_ref[0, k][:, :, None]
        out_ref[0, k] = jnp.concatenate([row_t, col_t, g], axis=-1)


def kernel(seq1M, seq2M, patches, geo):
    grid_spec = pltpu.PrefetchScalarGridSpec(
        num_scalar_prefetch=1,
        grid=(B, P // PP),
        in_specs=[
            pl.BlockSpec((1, SR, D), lambda i, p, s: (i, 0, 0)),
            pl.BlockSpec((1, SL, D), lambda i, p, s: (i, 0, 0)),
            pl.BlockSpec((1, PP, PS, PS), lambda i, p, s: (i, p, 0, 0)),
        ],
        out_specs=pl.BlockSpec(
            (1, PP, PS, PS, CH), lambda i, p, s: (i, p, 0, 0, 0)
        ),
    )
    return pl.pallas_call(
        _tc_body,
        grid_spec=grid_spec,
        out_shape=jax.ShapeDtypeStruct((B, P, PS, PS, CH), jnp.float32),
    )(patches, seq1M, seq2M, geo)


# speedup vs baseline: 3.3570x; 1.1982x over previous
import jax
import jax.numpy as jnp
from jax.experimental import pallas as pl
from jax.experimental.pallas import tpu as pltpu

B = 32
P = 16
PS = 32
D = 64
SR = 2048
SL = 1024
CH = 2 * D + 1


PP = 8


def _tc_body(patches_s, seq1_ref, seq2_ref, geo_ref, out_ref):
    i = pl.program_id(0)
    pb = pl.program_id(1)
    for k in range(PP):
        p = pb * PP + k
        r = patches_s[i, p, 0]
        c = patches_s[i, p, 1]
        row = seq1_ref[0, pl.ds(r, PS), :]
        col = seq2_ref[0, pl.ds(c, PS), :]
        row_t = jnp.broadcast_to(row[None, :, :], (PS, PS, D))
        col_t = jnp.broadcast_to(col[:, None, :], (PS, PS, D))
        g = geo_ref[0, k][:, :, None]
        out_ref[0, k] = jnp.concatenate([row_t, col_t, g], axis=-1)


def kernel(seq1M, seq2M, patches, geo):
    grid_spec = pltpu.PrefetchScalarGridSpec(
        num_scalar_prefetch=1,
        grid=(B, P // PP),
        in_specs=[
            pl.BlockSpec((1, SR, D), lambda i, p, s: (i, 0, 0)),
            pl.BlockSpec((1, SL, D), lambda i, p, s: (i, 0, 0)),
            pl.BlockSpec((1, PP, PS, PS), lambda i, p, s: (i, p, 0, 0)),
        ],
        out_specs=pl.BlockSpec(
            (1, PP, PS, PS, CH), lambda i, p, s: (i, p, 0, 0, 0)
        ),
    )
    return pl.pallas_call(
        _tc_body,
        grid_spec=grid_spec,
        out_shape=jax.ShapeDtypeStruct((B, P, PS, PS, CH), jnp.float32),
    )(patches, seq1M, seq2M, geo)


# speedup vs baseline: 3.4346x; 1.0231x over previous
import jax
import jax.numpy as jnp
from jax.experimental import pallas as pl
from jax.experimental.pallas import tpu as pltpu

B = 32
P = 16
PS = 32
D = 64
SR = 2048
SL = 1024
CH = 2 * D + 1


PP = 16


def _tc_body(patches_s, seq1_ref, seq2_ref, geo_ref, out_ref):
    i = pl.program_id(0)
    pb = pl.program_id(1)
    for k in range(PP):
        p = pb * PP + k
        r = patches_s[i, p, 0]
        c = patches_s[i, p, 1]
        row = seq1_ref[0, pl.ds(r, PS), :]
        col = seq2_ref[0, pl.ds(c, PS), :]
        row_t = jnp.broadcast_to(row[None, :, :], (PS, PS, D))
        col_t = jnp.broadcast_to(col[:, None, :], (PS, PS, D))
        g = geo_ref[0, k][:, :, None]
        out_ref[0, k] = jnp.concatenate([row_t, col_t, g], axis=-1)


def kernel(seq1M, seq2M, patches, geo):
    grid_spec = pltpu.PrefetchScalarGridSpec(
        num_scalar_prefetch=1,
        grid=(B, P // PP),
        in_specs=[
            pl.BlockSpec((1, SR, D), lambda i, p, s: (i, 0, 0)),
            pl.BlockSpec((1, SL, D), lambda i, p, s: (i, 0, 0)),
            pl.BlockSpec((1, PP, PS, PS), lambda i, p, s: (i, p, 0, 0)),
        ],
        out_specs=pl.BlockSpec(
            (1, PP, PS, PS, CH), lambda i, p, s: (i, p, 0, 0, 0)
        ),
    )
    return pl.pallas_call(
        _tc_body,
        grid_spec=grid_spec,
        out_shape=jax.ShapeDtypeStruct((B, P, PS, PS, CH), jnp.float32),
    )(patches, seq1M, seq2M, geo)
